# Initial kernel scaffold; baseline (speedup 1.0000x reference)
import jax
import jax.numpy as jnp
from jax.experimental import pallas as pl


def kernel(x, W, b):
    raise NotImplementedError("write your pallas kernel here")


# speedup vs baseline: 1.1604x; 1.1604x over previous
import functools

import jax
import jax.numpy as jnp
from jax.experimental import pallas as pl

N, D, E = 8192, 4096, 64
ROWS = 512


def _router_body(x_ref, w_ref, b_ref,
                 probs_ref, idx_ref, tprob_ref, aux_ref, me_ref, ce_ref,
                 ent_ref):
    i = pl.program_id(0)
    nsteps = pl.num_programs(0)

    logits = jnp.dot(x_ref[...], w_ref[...],
                     preferred_element_type=jnp.float32) + b_ref[...]
    m = jnp.max(logits, axis=-1, keepdims=True)
    ex = jnp.exp(logits - m)
    s = jnp.sum(ex, axis=-1, keepdims=True)
    p = ex / s
    probs_ref[...] = p

    idx = jnp.argmax(logits, axis=-1).astype(jnp.int32)
    idx_ref[...] = idx[None, None, :]
    tprob_ref[...] = jnp.max(p, axis=-1)[None, None, :]

    one_hot = (jax.lax.broadcasted_iota(jnp.int32, logits.shape, 1)
               == idx[:, None]).astype(jnp.float32)
    me_part = jnp.sum(one_hot, axis=0, keepdims=True) * (1.0 / N)
    ce_part = jnp.sum(p, axis=0, keepdims=True) * (1.0 / N)
    ent_rows = -jnp.sum(jnp.log(jnp.clip(p, 1e-9)) * p, axis=-1)
    ent_part = (jnp.sum(ent_rows) * (1.0 / N)).reshape(1, 1)

    @pl.when(i == 0)
    def _init():
        me_ref[...] = me_part
        ce_ref[...] = ce_part
        ent_ref[...] = ent_part

    @pl.when(i > 0)
    def _acc():
        me_ref[...] += me_part
        ce_ref[...] += ce_part
        ent_ref[...] += ent_part

    @pl.when(i == nsteps - 1)
    def _finish():
        aux_ref[...] = 0.05 * E * jnp.sum(
            me_ref[...] * ce_ref[...]).reshape(1, 1)


@functools.partial(jax.jit, static_argnames=())
def kernel(x, W, b):
    nsteps = N // ROWS
    b2 = b.reshape(1, E)
    out_types = (
        jax.ShapeDtypeStruct((N, E), jnp.float32),
        jax.ShapeDtypeStruct((nsteps, 1, ROWS), jnp.int32),
        jax.ShapeDtypeStruct((nsteps, 1, ROWS), jnp.float32),
        jax.ShapeDtypeStruct((1, 1), jnp.float32),
        jax.ShapeDtypeStruct((1, E), jnp.float32),
        jax.ShapeDtypeStruct((1, E), jnp.float32),
        jax.ShapeDtypeStruct((1, 1), jnp.float32),
    )
    grid_spec = pl.GridSpec(
        grid=(nsteps,),
        in_specs=[
            pl.BlockSpec((ROWS, D), lambda i: (i, 0)),
            pl.BlockSpec((D, E), lambda i: (0, 0)),
            pl.BlockSpec((1, E), lambda i: (0, 0)),
        ],
        out_specs=[
            pl.BlockSpec((ROWS, E), lambda i: (i, 0)),
            pl.BlockSpec((1, 1, ROWS), lambda i: (i, 0, 0)),
            pl.BlockSpec((1, 1, ROWS), lambda i: (i, 0, 0)),
            pl.BlockSpec((1, 1), lambda i: (0, 0)),
            pl.BlockSpec((1, E), lambda i: (0, 0)),
            pl.BlockSpec((1, E), lambda i: (0, 0)),
            pl.BlockSpec((1, 1), lambda i: (0, 0)),
        ],
    )
    probs, idx2, tp2, aux, me, ce, ent = pl.pallas_call(
        _router_body, grid_spec=grid_spec, out_shape=out_types)(x, W, b2)
    return (probs, idx2.reshape(N), tp2.reshape(N), aux[0, 0],
            me[0], ce[0], ent[0, 0])


# speedup vs baseline: 1.2385x; 1.0674x over previous
import functools

import jax
import jax.numpy as jnp
from jax.experimental import pallas as pl

N, D, E = 8192, 4096, 64
ROWS = 1024


def _router_body(x_ref, w_ref, b_ref,
                 probs_ref, idx_ref, tprob_ref, aux_ref, me_ref, ce_ref,
                 ent_ref):
    i = pl.program_id(0)
    nsteps = pl.num_programs(0)

    logits = jnp.dot(x_ref[...], w_ref[...],
                     preferred_element_type=jnp.float32) + b_ref[...]
    m = jnp.max(logits, axis=-1, keepdims=True)
    ex = jnp.exp(logits - m)
    s = jnp.sum(ex, axis=-1, keepdims=True)
    p = ex / s
    probs_ref[...] = p

    idx = jnp.argmax(logits, axis=-1).astype(jnp.int32)
    idx_ref[...] = idx[None, None, :]
    tprob_ref[...] = jnp.max(p, axis=-1)[None, None, :]

    one_hot = (jax.lax.broadcasted_iota(jnp.int32, logits.shape, 1)
               == idx[:, None]).astype(jnp.float32)
    me_part = jnp.sum(one_hot, axis=0, keepdims=True) * (1.0 / N)
    ce_part = jnp.sum(p, axis=0, keepdims=True) * (1.0 / N)
    ent_rows = -jnp.sum(jnp.log(jnp.clip(p, 1e-9)) * p, axis=-1)
    ent_part = (jnp.sum(ent_rows) * (1.0 / N)).reshape(1, 1)

    @pl.when(i == 0)
    def _init():
        me_ref[...] = me_part
        ce_ref[...] = ce_part
        ent_ref[...] = ent_part

    @pl.when(i > 0)
    def _acc():
        me_ref[...] += me_part
        ce_ref[...] += ce_part
        ent_ref[...] += ent_part

    @pl.when(i == nsteps - 1)
    def _finish():
        aux_ref[...] = 0.05 * E * jnp.sum(
            me_ref[...] * ce_ref[...]).reshape(1, 1)


@functools.partial(jax.jit, static_argnames=())
def kernel(x, W, b):
    nsteps = N // ROWS
    b2 = b.reshape(1, E)
    out_types = (
        jax.ShapeDtypeStruct((N, E), jnp.float32),
        jax.ShapeDtypeStruct((nsteps, 1, ROWS), jnp.int32),
        jax.ShapeDtypeStruct((nsteps, 1, ROWS), jnp.float32),
        jax.ShapeDtypeStruct((1, 1), jnp.float32),
        jax.ShapeDtypeStruct((1, E), jnp.float32),
        jax.ShapeDtypeStruct((1, E), jnp.float32),
        jax.ShapeDtypeStruct((1, 1), jnp.float32),
    )
    grid_spec = pl.GridSpec(
        grid=(nsteps,),
        in_specs=[
            pl.BlockSpec((ROWS, D), lambda i: (i, 0)),
            pl.BlockSpec((D, E), lambda i: (0, 0)),
            pl.BlockSpec((1, E), lambda i: (0, 0)),
        ],
        out_specs=[
            pl.BlockSpec((ROWS, E), lambda i: (i, 0)),
            pl.BlockSpec((1, 1, ROWS), lambda i: (i, 0, 0)),
            pl.BlockSpec((1, 1, ROWS), lambda i: (i, 0, 0)),
            pl.BlockSpec((1, 1), lambda i: (0, 0)),
            pl.BlockSpec((1, E), lambda i: (0, 0)),
            pl.BlockSpec((1, E), lambda i: (0, 0)),
            pl.BlockSpec((1, 1), lambda i: (0, 0)),
        ],
    )
    probs, idx2, tp2, aux, me, ce, ent = pl.pallas_call(
        _router_body, grid_spec=grid_spec, out_shape=out_types)(x, W, b2)
    return (probs, idx2.reshape(N), tp2.reshape(N), aux[0, 0],
            me[0], ce[0], ent[0, 0])


# speedup vs baseline: 1.2391x; 1.0004x over previous
import functools

import jax
import jax.numpy as jnp
from jax.experimental import pallas as pl

N, D, E = 8192, 4096, 64
ROWS = 1024


def _router_body(x_ref, w_ref, b_ref,
                 probs_ref, idx_ref, tprob_ref, aux_ref, me_ref, ce_ref,
                 ent_ref):
    i = pl.program_id(0)
    nsteps = pl.num_programs(0)

    logits = jnp.dot(x_ref[...], w_ref[...],
                     preferred_element_type=jnp.float32) + b_ref[...]
    m = jnp.max(logits, axis=-1, keepdims=True)
    d = logits - m
    ex = jnp.exp(d)
    s = jnp.sum(ex, axis=-1, keepdims=True)
    rinv = 1.0 / s
    p = ex * rinv
    probs_ref[...] = p

    lane = jax.lax.broadcasted_iota(jnp.int32, logits.shape, 1)
    idx = jnp.min(jnp.where(d >= 0.0, lane, jnp.int32(E)), axis=-1)
    idx_ref[...] = idx[None, None, :]
    tprob_ref[...] = rinv[None, None, :, 0]

    one_hot = (lane == idx[:, None]).astype(jnp.float32)
    me_part = jnp.sum(one_hot, axis=0, keepdims=True) * (1.0 / N)
    ce_part = jnp.sum(p, axis=0, keepdims=True) * (1.0 / N)
    ent_rows = jnp.log(s[:, 0]) - jnp.sum(p * d, axis=-1)
    ent_part = (jnp.sum(ent_rows) * (1.0 / N)).reshape(1, 1)

    @pl.when(i == 0)
    def _init():
        me_ref[...] = me_part
        ce_ref[...] = ce_part
        ent_ref[...] = ent_part

    @pl.when(i > 0)
    def _acc():
        me_ref[...] += me_part
        ce_ref[...] += ce_part
        ent_ref[...] += ent_part

    @pl.when(i == nsteps - 1)
    def _finish():
        aux_ref[...] = 0.05 * E * jnp.sum(
            me_ref[...] * ce_ref[...]).reshape(1, 1)


@functools.partial(jax.jit, static_argnames=())
def kernel(x, W, b):
    nsteps = N // ROWS
    b2 = b.reshape(1, E)
    out_types = (
        jax.ShapeDtypeStruct((N, E), jnp.float32),
        jax.ShapeDtypeStruct((nsteps, 1, ROWS), jnp.int32),
        jax.ShapeDtypeStruct((nsteps, 1, ROWS), jnp.float32),
        jax.ShapeDtypeStruct((1, 1), jnp.float32),
        jax.ShapeDtypeStruct((1, E), jnp.float32),
        jax.ShapeDtypeStruct((1, E), jnp.float32),
        jax.ShapeDtypeStruct((1, 1), jnp.float32),
    )
    grid_spec = pl.GridSpec(
        grid=(nsteps,),
        in_specs=[
            pl.BlockSpec((ROWS, D), lambda i: (i, 0)),
            pl.BlockSpec((D, E), lambda i: (0, 0)),
            pl.BlockSpec((1, E), lambda i: (0, 0)),
        ],
        out_specs=[
            pl.BlockSpec((ROWS, E), lambda i: (i, 0)),
            pl.BlockSpec((1, 1, ROWS), lambda i: (i, 0, 0)),
            pl.BlockSpec((1, 1, ROWS), lambda i: (i, 0, 0)),
            pl.BlockSpec((1, 1), lambda i: (0, 0)),
            pl.BlockSpec((1, E), lambda i: (0, 0)),
            pl.BlockSpec((1, E), lambda i: (0, 0)),
            pl.BlockSpec((1, 1), lambda i: (0, 0)),
        ],
    )
    probs, idx2, tp2, aux, me, ce, ent = pl.pallas_call(
        _router_body, grid_spec=grid_spec, out_shape=out_types)(x, W, b2)
    return (probs, idx2.reshape(N), tp2.reshape(N), aux[0, 0],
            me[0], ce[0], ent[0, 0])
